# Initial kernel scaffold; baseline (speedup 1.0000x reference)
import jax
import jax.numpy as jnp
from jax.experimental import pallas as pl


def kernel(x, edge_index, W1, b1, W2, b2):
    raise NotImplementedError("write your pallas kernel here")


# speedup vs baseline: 10.7261x; 10.7261x over previous
import functools

import jax
import jax.numpy as jnp
from jax import lax
from jax.experimental import pallas as pl
from jax.experimental.pallas import tpu as pltpu
from jax.experimental.pallas import tpu_sc as plsc

N = 10000
E = 320000
IN_DIM = 128
HID_DIM = 128
OUT_DIM = 64

NC = 2
NS = 16
NW = NC * NS
CH = 128
ROWS_PER_SUB = 632
NPAD = NS * ROWS_PER_SUB
STEPS = (E + NW * CH - 1) // (NW * CH)
EPW = STEPS * CH
EPAD = EPW * NW


def _vmesh():
    return plsc.VectorSubcoreMesh(core_axis_name="c", subcore_axis_name="s")


def _make_edge_agg(D):
    @functools.partial(
        pl.kernel,
        out_type=jax.ShapeDtypeStruct((NC, NPAD, D), jnp.float32),
        mesh=_vmesh(),
        scratch_types=[
            pltpu.VMEM((CH,), jnp.int32),
            pltpu.VMEM((CH,), jnp.int32),
            pltpu.VMEM((CH, D), jnp.float32),
            pltpu.VMEM_SHARED((NPAD, D), jnp.float32),
            pltpu.SemaphoreType.DMA,
        ],
    )
    def agg(h_hbm, src_hbm, dst_hbm, zer_hbm, out_hbm,
            src_v, dst_v, rows_v, acc_sh, sem):
        cid = lax.axis_index("c")
        sid = lax.axis_index("s")
        wid = sid * NC + cid
        row0 = sid * ROWS_PER_SUB
        pltpu.sync_copy(zer_hbm, acc_sh.at[pl.ds(row0, ROWS_PER_SUB)])
        plsc.subcore_barrier()
        base = wid * EPW

        @pl.loop(0, STEPS)
        def _(step):
            off = pl.multiple_of(base + step * CH, CH)
            pltpu.sync_copy(src_hbm.at[pl.ds(off, CH)], src_v)
            pltpu.sync_copy(dst_hbm.at[pl.ds(off, CH)], dst_v)
            pltpu.async_copy(h_hbm.at[src_v], rows_v, sem).wait()
            pltpu.sync_copy(rows_v, acc_sh.at[dst_v], add=True)

        plsc.subcore_barrier()
        pltpu.sync_copy(acc_sh.at[pl.ds(row0, ROWS_PER_SUB)],
                        out_hbm.at[cid].at[pl.ds(row0, ROWS_PER_SUB)])

    return agg


_agg_hid = _make_edge_agg(HID_DIM)


@functools.partial(
    pl.kernel,
    out_type=jax.ShapeDtypeStruct((NC, NPAD, HID_DIM), jnp.float32),
    mesh=_vmesh(),
    scratch_types=[
        pltpu.VMEM((CH,), jnp.int32),
        pltpu.VMEM((CH, HID_DIM), jnp.float32),
        pltpu.VMEM_SHARED((NPAD, HID_DIM), jnp.float32),
    ],
)
def _deg_kernel(dst_hbm, ones_hbm, zer_hbm, out_hbm, dst_v, ones_v, acc_sh):
    cid = lax.axis_index("c")
    sid = lax.axis_index("s")
    wid = sid * NC + cid
    row0 = sid * ROWS_PER_SUB
    pltpu.sync_copy(ones_hbm, ones_v)
    pltpu.sync_copy(zer_hbm, acc_sh.at[pl.ds(row0, ROWS_PER_SUB)])
    plsc.subcore_barrier()
    base = wid * EPW

    @pl.loop(0, STEPS)
    def _(step):
        off = pl.multiple_of(base + step * CH, CH)
        pltpu.sync_copy(dst_hbm.at[pl.ds(off, CH)], dst_v)
        pltpu.sync_copy(ones_v, acc_sh.at[dst_v], add=True)

    plsc.subcore_barrier()
    pltpu.sync_copy(acc_sh.at[pl.ds(row0, ROWS_PER_SUB)],
                    out_hbm.at[cid].at[pl.ds(row0, ROWS_PER_SUB)])


def _dot(a, b):
    return lax.dot_general(a, b, (((1,), (0,)), ((), ())),
                           precision=lax.Precision.HIGHEST,
                           preferred_element_type=jnp.float32)


def _mm1_body(x_ref, w_ref, o_ref):
    o_ref[...] = _dot(x_ref[...], w_ref[...])


_mm1 = pl.pallas_call(
    _mm1_body,
    out_shape=jax.ShapeDtypeStruct((NPAD, HID_DIM), jnp.float32),
)


def _prep_body(h_ref, degp_ref, dinv_ref, h1p_ref):
    deg = degp_ref[0, :, 0:1] + degp_ref[1, :, 0:1] + 1.0
    dinv = jnp.broadcast_to(lax.rsqrt(deg), (NPAD, HID_DIM))
    dinv_ref[...] = dinv
    h1p_ref[...] = h_ref[...] * dinv


_prep = pl.pallas_call(
    _prep_body,
    out_shape=(
        jax.ShapeDtypeStruct((NPAD, HID_DIM), jnp.float32),
        jax.ShapeDtypeStruct((NPAD, HID_DIM), jnp.float32),
    ),
)


def _mid_body(acc_ref, h1p_ref, dinv_ref, b1_ref, o_ref):
    z = (acc_ref[0] + acc_ref[1] + h1p_ref[...]) * dinv_ref[...] + b1_ref[...]
    o_ref[...] = jnp.maximum(z, 0.0) * dinv_ref[...]


_mid = pl.pallas_call(
    _mid_body,
    out_shape=jax.ShapeDtypeStruct((NPAD, HID_DIM), jnp.float32),
)


def _out_body(acc_ref, u_ref, dinv_ref, b2_ref, w2_ref, o_ref):
    v = (acc_ref[0] + acc_ref[1] + u_ref[...]) * dinv_ref[...]
    o_ref[...] = (_dot(v, w2_ref[...]) + b2_ref[...])[:N]


_outk = pl.pallas_call(
    _out_body,
    out_shape=jax.ShapeDtypeStruct((N, OUT_DIM), jnp.float32),
)


def kernel(x, edge_index, W1, b1, W2, b2):
    src = edge_index[0].astype(jnp.int32)
    dst = edge_index[1].astype(jnp.int32)
    pad = jnp.full((EPAD - E,), N, jnp.int32)
    src_p = jnp.concatenate([src, pad])
    dst_p = jnp.concatenate([dst, pad])
    x_pad = jnp.pad(x, ((0, NPAD - N), (0, 0)))

    zer_h = jnp.zeros((ROWS_PER_SUB, HID_DIM), jnp.float32)
    ones_d = jnp.ones((CH, HID_DIM), jnp.float32)

    degp = _deg_kernel(dst_p, ones_d, zer_h)
    h1 = _mm1(x_pad, W1)
    dinv, h1p = _prep(h1, degp)
    acc1 = _agg_hid(h1p, src_p, dst_p, zer_h)
    u = _mid(acc1, h1p, dinv, b1.reshape(1, HID_DIM))
    acc2 = _agg_hid(u, src_p, dst_p, zer_h)
    return _outk(acc2, u, dinv, b2.reshape(1, OUT_DIM), W2)
